# baseline (speedup 1.0000x reference)
import functools
import numpy as np
import jax
import jax.numpy as jnp
from jax import lax
from jax.experimental import pallas as pl
from jax.experimental.pallas import tpu as pltpu
from jax.experimental.pallas import tpu_sc as plsc

N = 10000
E = 320000
D = 128
B = 16
NPG = N // B
K = 313
OUT_DIM = 256
NC = 2
NS = 16
NW = NC * NS
EPW = E // NW

_HIGH = lax.Precision.HIGHEST


GRID1 = 5
XB = 2048
EB = E // GRID1
SPAD = GRID1 * XB


def _prep_body(x_ref, w2_ref, br_ref, s_ref, r_ref):
    sr = lax.dot_general(x_ref[...], w2_ref[...],
                         (((1,), (1,)), ((), ())), precision=_HIGH)
    s_ref[...] = sr[:, 0]
    r_ref[...] = sr[:, 1] + br_ref[0]


def _prep(x, w_rel, w_root, b_rel):
    w2 = jnp.concatenate([w_rel, w_root], axis=0)
    return pl.pallas_call(
        _prep_body,
        grid=(GRID1,),
        in_specs=[
            pl.BlockSpec((XB, D), lambda i: (i, 0)),
            pl.BlockSpec((2, D), lambda i: (0, 0)),
            pl.BlockSpec(memory_space=pltpu.SMEM),
        ],
        out_specs=[
            pl.BlockSpec((XB,), lambda i: (i,)),
            pl.BlockSpec((XB,), lambda i: (i,)),
        ],
        out_shape=[
            jax.ShapeDtypeStruct((SPAD,), jnp.float32),
            jax.ShapeDtypeStruct((SPAD,), jnp.float32),
        ],
    )(x, w2, b_rel)


_sc_mesh = plsc.VectorSubcoreMesh(core_axis_name="c", subcore_axis_name="s")


TILE = 128
NT = E // TILE
NT_MAIN = NT // NW
EXTRA = NT - NW * NT_MAIN
EMAIN = NT_MAIN * TILE
ECHUNK = EMAIN + TILE


@functools.partial(
    pl.kernel,
    out_type=jax.ShapeDtypeStruct((NW, N), jnp.float32),
    mesh=_sc_mesh,
    scratch_types=[
        pltpu.VMEM((SPAD,), jnp.float32),
        pltpu.VMEM((2, ECHUNK), jnp.int32),
        pltpu.VMEM((N,), jnp.float32),
    ],
    compiler_params=pltpu.CompilerParams(needs_layout_passes=False),
)
def _seg_sum(s_hbm, r_hbm, ei_hbm, out_hbm, s_v, ei_v, acc_v):
    wid = lax.axis_index("s") * NC + lax.axis_index("c")
    pltpu.sync_copy(s_hbm, s_v)
    pltpu.sync_copy(ei_hbm.at[:, pl.ds(wid * EMAIN, EMAIN)],
                    ei_v.at[:, pl.ds(0, EMAIN)])

    @pl.when(wid == 0)
    def _():
        pltpu.sync_copy(r_hbm.at[pl.ds(0, N)], acc_v)

    @pl.when(wid != 0)
    def _():
        zeros16 = jnp.zeros((16,), jnp.float32)

        @plsc.parallel_loop(0, N // 16, unroll=8)
        def _zero(i):
            acc_v[pl.ds(i * 16, 16)] = zeros16

    @pl.when(wid < EXTRA)
    def _():
        pltpu.sync_copy(ei_hbm.at[:, pl.ds((NW * NT_MAIN + wid) * TILE, TILE)],
                        ei_v.at[:, pl.ds(EMAIN, TILE)])

    @plsc.parallel_loop(0, EMAIN // 16, unroll=8)
    def _edges(i):
        si = ei_v[0, pl.ds(i * 16, 16)]
        di = ei_v[1, pl.ds(i * 16, 16)]
        vals = plsc.load_gather(s_v, [si])
        plsc.addupdate_scatter(acc_v, [di], vals)

    @pl.when(wid < EXTRA)
    def _():
        @plsc.parallel_loop(EMAIN // 16, ECHUNK // 16, unroll=8)
        def _edges_extra(i):
            si = ei_v[0, pl.ds(i * 16, 16)]
            di = ei_v[1, pl.ds(i * 16, 16)]
            vals = plsc.load_gather(s_v, [si])
            plsc.addupdate_scatter(acc_v, [di], vals)

    pltpu.sync_copy(acc_v, out_hbm.at[wid])


def _pool_body(x_ref, p_ref, pw_ref, pb_ref, o_ref):
    x3 = x_ref[...]
    aggr = jnp.sum(p_ref[...], axis=0)
    z = jnp.stack([lax.slice(aggr, (b * NPG,), ((b + 1) * NPG,))
                   for b in range(B)], axis=0)
    score = jnp.tanh(z)

    bits = lax.bitcast_convert_type(score, jnp.int32)
    key = jnp.where(bits >= 0, bits, bits ^ jnp.int32(0x7FFFFFFF))
    int_min = jnp.int32(-(2 ** 31))

    tu = jnp.zeros((B, 1), jnp.int32)
    for bit in range(31, -1, -1):
        v = 1 << bit
        if v >= 2 ** 31:
            v -= 2 ** 32
        cand_u = tu | jnp.int32(v)
        cand_s = cand_u ^ int_min
        cnt = jnp.sum((key >= cand_s).astype(jnp.int32), axis=1, keepdims=True)
        tu = jnp.where(cnt >= K, cand_u, tu)
    ts = tu ^ int_min

    gt = key > ts
    eq = key == ts
    cnt_gt = jnp.sum(gt.astype(jnp.int32), axis=1, keepdims=True)
    ii = lax.broadcasted_iota(jnp.int32, (NPG, NPG), 0)
    jj = lax.broadcasted_iota(jnp.int32, (NPG, NPG), 1)
    tri = (ii <= jj).astype(jnp.float32)
    eq_rank = lax.dot_general(
        eq.astype(jnp.float32), tri,
        (((1,), (0,)), ((), ())), precision=_HIGH)
    need = (K - cnt_gt).astype(jnp.float32)
    mask = gt | (eq & (eq_rank <= need))

    w = jnp.where(mask, score, jnp.float32(0.0))
    pooled = lax.dot_general(
        w, x3, (((1,), (1,)), ((0,), (0,))), precision=_HIGH)
    pooled = pooled * jnp.float32(1.0 / K)
    out = lax.dot_general(
        pooled, pw_ref[...], (((1,), (1,)), ((), ())), precision=_HIGH)
    o_ref[...] = out + pb_ref[...].reshape(1, OUT_DIM)


def _pool_project(x3, p3, proj_w, proj_b):
    return pl.pallas_call(
        _pool_body,
        out_shape=jax.ShapeDtypeStruct((B, OUT_DIM), jnp.float32),
        in_specs=[
            pl.BlockSpec(memory_space=pltpu.VMEM),
            pl.BlockSpec(memory_space=pltpu.VMEM),
            pl.BlockSpec(memory_space=pltpu.VMEM),
            pl.BlockSpec(memory_space=pltpu.VMEM),
        ],
    )(x3, p3, proj_w, proj_b)


def kernel(x, edge_index, batch, gnn_w_rel, gnn_b_rel, gnn_w_root, proj_w, proj_b):
    s, r = _prep(x, gnn_w_rel, gnn_w_root, gnn_b_rel)
    partials = _seg_sum(s, r, edge_index)
    x3 = x.reshape(B, NPG, D)
    return _pool_project(x3, partials, proj_w, proj_b)


# speedup vs baseline: 1.1731x; 1.1731x over previous
import functools
import numpy as np
import jax
import jax.numpy as jnp
from jax import lax
from jax.experimental import pallas as pl
from jax.experimental.pallas import tpu as pltpu
from jax.experimental.pallas import tpu_sc as plsc

N = 10000
E = 320000
D = 128
B = 16
NPG = N // B
K = 313
OUT_DIM = 256
NC = 2
NS = 16
NW = NC * NS
EPW = E // NW

_HIGH = lax.Precision.HIGHEST


GRID1 = 5
XB = 2048
EB = E // GRID1
SPAD = GRID1 * XB


def _prep_body(x_ref, w2_ref, br_ref, s_ref, r_ref):
    sr = lax.dot_general(w2_ref[...], x_ref[...],
                         (((1,), (1,)), ((), ())), precision=_HIGH)
    s_ref[...] = sr[0]
    r_ref[...] = sr[1] + br_ref[0]


def _prep(x, w_rel, w_root, b_rel):
    w2 = jnp.concatenate([w_rel, w_root], axis=0)
    return pl.pallas_call(
        _prep_body,
        grid=(GRID1,),
        in_specs=[
            pl.BlockSpec((XB, D), lambda i: (i, 0)),
            pl.BlockSpec((2, D), lambda i: (0, 0)),
            pl.BlockSpec(memory_space=pltpu.SMEM),
        ],
        out_specs=[
            pl.BlockSpec((XB,), lambda i: (i,)),
            pl.BlockSpec((XB,), lambda i: (i,)),
        ],
        out_shape=[
            jax.ShapeDtypeStruct((SPAD,), jnp.float32),
            jax.ShapeDtypeStruct((SPAD,), jnp.float32),
        ],
    )(x, w2, b_rel)


_sc_mesh = plsc.VectorSubcoreMesh(core_axis_name="c", subcore_axis_name="s")


TILE = 128
NT = E // TILE
NT_MAIN = NT // NW
EXTRA = NT - NW * NT_MAIN
EMAIN = NT_MAIN * TILE
ECHUNK = EMAIN + TILE


@functools.partial(
    pl.kernel,
    out_type=jax.ShapeDtypeStruct((NW, N), jnp.float32),
    mesh=_sc_mesh,
    scratch_types=[
        pltpu.VMEM((SPAD,), jnp.float32),
        pltpu.VMEM((2, ECHUNK), jnp.int32),
        pltpu.VMEM((N,), jnp.float32),
    ],
    compiler_params=pltpu.CompilerParams(needs_layout_passes=False),
)
def _seg_sum(s_hbm, r_hbm, ei_hbm, out_hbm, s_v, ei_v, acc_v):
    wid = lax.axis_index("s") * NC + lax.axis_index("c")
    pltpu.sync_copy(s_hbm, s_v)
    pltpu.sync_copy(ei_hbm.at[:, pl.ds(wid * EMAIN, EMAIN)],
                    ei_v.at[:, pl.ds(0, EMAIN)])

    @pl.when(wid == 0)
    def _():
        pltpu.sync_copy(r_hbm.at[pl.ds(0, N)], acc_v)

    @pl.when(wid != 0)
    def _():
        zeros16 = jnp.zeros((16,), jnp.float32)

        @plsc.parallel_loop(0, N // 16, unroll=8)
        def _zero(i):
            acc_v[pl.ds(i * 16, 16)] = zeros16

    @pl.when(wid < EXTRA)
    def _():
        pltpu.sync_copy(ei_hbm.at[:, pl.ds((NW * NT_MAIN + wid) * TILE, TILE)],
                        ei_v.at[:, pl.ds(EMAIN, TILE)])

    @plsc.parallel_loop(0, EMAIN // 16, unroll=8)
    def _edges(i):
        si = ei_v[0, pl.ds(i * 16, 16)]
        di = ei_v[1, pl.ds(i * 16, 16)]
        vals = plsc.load_gather(s_v, [si])
        plsc.addupdate_scatter(acc_v, [di], vals)

    @pl.when(wid < EXTRA)
    def _():
        @plsc.parallel_loop(EMAIN // 16, ECHUNK // 16, unroll=8)
        def _edges_extra(i):
            si = ei_v[0, pl.ds(i * 16, 16)]
            di = ei_v[1, pl.ds(i * 16, 16)]
            vals = plsc.load_gather(s_v, [si])
            plsc.addupdate_scatter(acc_v, [di], vals)

    pltpu.sync_copy(acc_v, out_hbm.at[wid])


def _pool_body(x_ref, p_ref, pw_ref, pb_ref, o_ref):
    x3 = x_ref[...]
    aggr = jnp.sum(p_ref[...], axis=0)
    z = jnp.stack([lax.slice(aggr, (b * NPG,), ((b + 1) * NPG,))
                   for b in range(B)], axis=0)
    score = jnp.tanh(z)

    bits = lax.bitcast_convert_type(score, jnp.int32)
    key = jnp.where(bits >= 0, bits, bits ^ jnp.int32(0x7FFFFFFF))
    int_min = jnp.int32(-(2 ** 31))

    tu = jnp.zeros((B, 1), jnp.int32)
    for bit in range(31, -1, -1):
        v = 1 << bit
        if v >= 2 ** 31:
            v -= 2 ** 32
        cand_u = tu | jnp.int32(v)
        cand_s = cand_u ^ int_min
        cnt = jnp.sum((key >= cand_s).astype(jnp.int32), axis=1, keepdims=True)
        tu = jnp.where(cnt >= K, cand_u, tu)
    ts = tu ^ int_min

    gt = key > ts
    eq = key == ts
    cnt_gt = jnp.sum(gt.astype(jnp.int32), axis=1, keepdims=True)
    ii = lax.broadcasted_iota(jnp.int32, (NPG, NPG), 0)
    jj = lax.broadcasted_iota(jnp.int32, (NPG, NPG), 1)
    tri = (ii <= jj).astype(jnp.float32)
    eq_rank = lax.dot_general(
        eq.astype(jnp.float32), tri,
        (((1,), (0,)), ((), ())), precision=_HIGH)
    need = (K - cnt_gt).astype(jnp.float32)
    mask = gt | (eq & (eq_rank <= need))

    w = jnp.where(mask, score, jnp.float32(0.0))
    pooled = lax.dot_general(
        w, x3, (((1,), (1,)), ((0,), (0,))), precision=_HIGH)
    pooled = pooled * jnp.float32(1.0 / K)
    out = lax.dot_general(
        pooled, pw_ref[...], (((1,), (1,)), ((), ())), precision=_HIGH)
    o_ref[...] = out + pb_ref[...].reshape(1, OUT_DIM)


def _pool_project(x3, p3, proj_w, proj_b):
    return pl.pallas_call(
        _pool_body,
        out_shape=jax.ShapeDtypeStruct((B, OUT_DIM), jnp.float32),
        in_specs=[
            pl.BlockSpec(memory_space=pltpu.VMEM),
            pl.BlockSpec(memory_space=pltpu.VMEM),
            pl.BlockSpec(memory_space=pltpu.VMEM),
            pl.BlockSpec(memory_space=pltpu.VMEM),
        ],
    )(x3, p3, proj_w, proj_b)


def kernel(x, edge_index, batch, gnn_w_rel, gnn_b_rel, gnn_w_root, proj_w, proj_b):
    s, r = _prep(x, gnn_w_rel, gnn_w_root, gnn_b_rel)
    partials = _seg_sum(s, r, edge_index)
    x3 = x.reshape(B, NPG, D)
    return _pool_project(x3, partials, proj_w, proj_b)


# speedup vs baseline: 1.1834x; 1.0088x over previous
import functools
import numpy as np
import jax
import jax.numpy as jnp
from jax import lax
from jax.experimental import pallas as pl
from jax.experimental.pallas import tpu as pltpu
from jax.experimental.pallas import tpu_sc as plsc

N = 10000
E = 320000
D = 128
B = 16
NPG = N // B
K = 313
OUT_DIM = 256
NC = 2
NS = 16
NW = NC * NS
EPW = E // NW

_HIGH = lax.Precision.HIGHEST


GRID1 = 10
XB = 1024
SPAD = GRID1 * XB


def _prep_body(x_ref, wrel_ref, wroot_ref, br_ref, s_ref, r_ref):
    w2 = jnp.concatenate([wrel_ref[...], wroot_ref[...]], axis=0)
    sr = lax.dot_general(w2, x_ref[...],
                         (((1,), (1,)), ((), ())), precision=_HIGH)
    s_ref[...] = sr[0]
    r_ref[...] = sr[1] + br_ref[0]


def _prep(x, w_rel, w_root, b_rel):
    return pl.pallas_call(
        _prep_body,
        grid=(GRID1,),
        in_specs=[
            pl.BlockSpec((XB, D), lambda i: (i, 0)),
            pl.BlockSpec((1, D), lambda i: (0, 0)),
            pl.BlockSpec((1, D), lambda i: (0, 0)),
            pl.BlockSpec(memory_space=pltpu.SMEM),
        ],
        out_specs=[
            pl.BlockSpec((XB,), lambda i: (i,)),
            pl.BlockSpec((XB,), lambda i: (i,)),
        ],
        out_shape=[
            jax.ShapeDtypeStruct((SPAD,), jnp.float32),
            jax.ShapeDtypeStruct((SPAD,), jnp.float32),
        ],
    )(x, w_rel, w_root, b_rel)


_sc_mesh = plsc.VectorSubcoreMesh(core_axis_name="c", subcore_axis_name="s")


TILE = 128
NT = E // TILE
NT_MAIN = NT // NW
EXTRA = NT - NW * NT_MAIN
EMAIN = NT_MAIN * TILE
ECHUNK = EMAIN + TILE


@functools.partial(
    pl.kernel,
    out_type=jax.ShapeDtypeStruct((NW, N), jnp.float32),
    mesh=_sc_mesh,
    scratch_types=[
        pltpu.VMEM((SPAD,), jnp.float32),
        pltpu.VMEM((2, ECHUNK), jnp.int32),
        pltpu.VMEM((N,), jnp.float32),
    ],
    compiler_params=pltpu.CompilerParams(needs_layout_passes=False),
)
def _seg_sum(s_hbm, r_hbm, ei_hbm, out_hbm, s_v, ei_v, acc_v):
    wid = lax.axis_index("s") * NC + lax.axis_index("c")
    pltpu.sync_copy(s_hbm, s_v)
    pltpu.sync_copy(ei_hbm.at[:, pl.ds(wid * EMAIN, EMAIN)],
                    ei_v.at[:, pl.ds(0, EMAIN)])

    @pl.when(wid == 0)
    def _():
        pltpu.sync_copy(r_hbm.at[pl.ds(0, N)], acc_v)

    @pl.when(wid != 0)
    def _():
        zeros16 = jnp.zeros((16,), jnp.float32)

        @plsc.parallel_loop(0, N // 16, unroll=16)
        def _zero(i):
            acc_v[pl.ds(i * 16, 16)] = zeros16

    @pl.when(wid < EXTRA)
    def _():
        pltpu.sync_copy(ei_hbm.at[:, pl.ds((NW * NT_MAIN + wid) * TILE, TILE)],
                        ei_v.at[:, pl.ds(EMAIN, TILE)])

    @plsc.parallel_loop(0, EMAIN // 16, unroll=16)
    def _edges(i):
        si = ei_v[0, pl.ds(i * 16, 16)]
        di = ei_v[1, pl.ds(i * 16, 16)]
        vals = plsc.load_gather(s_v, [si])
        plsc.addupdate_scatter(acc_v, [di], vals)

    @pl.when(wid < EXTRA)
    def _():
        @plsc.parallel_loop(EMAIN // 16, ECHUNK // 16, unroll=16)
        def _edges_extra(i):
            si = ei_v[0, pl.ds(i * 16, 16)]
            di = ei_v[1, pl.ds(i * 16, 16)]
            vals = plsc.load_gather(s_v, [si])
            plsc.addupdate_scatter(acc_v, [di], vals)

    pltpu.sync_copy(acc_v, out_hbm.at[wid])


def _pool_body(x_ref, p_ref, pw_ref, pb_ref, o_ref, xv_ref, sem):
    xcopy = pltpu.make_async_copy(x_ref, xv_ref, sem)
    xcopy.start()
    aggr = jnp.sum(p_ref[...], axis=0)
    z = jnp.stack([lax.slice(aggr, (b * NPG,), ((b + 1) * NPG,))
                   for b in range(B)], axis=0)
    score = jnp.tanh(z)

    bits = lax.bitcast_convert_type(score, jnp.int32)
    key = jnp.where(bits >= 0, bits, bits ^ jnp.int32(0x7FFFFFFF))
    int_min = jnp.int32(-(2 ** 31))

    tu = jnp.zeros((B, 1), jnp.int32)
    for bit in range(31, -1, -1):
        v = 1 << bit
        if v >= 2 ** 31:
            v -= 2 ** 32
        cand_u = tu | jnp.int32(v)
        cand_s = cand_u ^ int_min
        cnt = jnp.sum((key >= cand_s).astype(jnp.int32), axis=1, keepdims=True)
        tu = jnp.where(cnt >= K, cand_u, tu)
    ts = tu ^ int_min

    gt = key > ts
    eq = key == ts
    cnt_gt = jnp.sum(gt.astype(jnp.int32), axis=1, keepdims=True)
    ii = lax.broadcasted_iota(jnp.int32, (NPG, NPG), 0)
    jj = lax.broadcasted_iota(jnp.int32, (NPG, NPG), 1)
    tri = (ii <= jj).astype(jnp.float32)
    eq_rank = lax.dot_general(
        eq.astype(jnp.float32), tri,
        (((1,), (0,)), ((), ())), precision=_HIGH)
    need = (K - cnt_gt).astype(jnp.float32)
    mask = gt | (eq & (eq_rank <= need))

    w = jnp.where(mask, score, jnp.float32(0.0))
    xcopy.wait()
    pooled = lax.dot_general(
        w, xv_ref[...], (((1,), (1,)), ((0,), (0,))), precision=_HIGH)
    pooled = pooled * jnp.float32(1.0 / K)
    out = lax.dot_general(
        pooled, pw_ref[...], (((1,), (1,)), ((), ())), precision=_HIGH)
    o_ref[...] = out + pb_ref[...].reshape(1, OUT_DIM)


def _pool_project(x3, p3, proj_w, proj_b):
    return pl.pallas_call(
        _pool_body,
        out_shape=jax.ShapeDtypeStruct((B, OUT_DIM), jnp.float32),
        in_specs=[
            pl.BlockSpec(memory_space=pl.ANY),
            pl.BlockSpec(memory_space=pltpu.VMEM),
            pl.BlockSpec(memory_space=pltpu.VMEM),
            pl.BlockSpec(memory_space=pltpu.VMEM),
        ],
        scratch_shapes=[
            pltpu.VMEM((B, NPG, D), jnp.float32),
            pltpu.SemaphoreType.DMA,
        ],
    )(x3, p3, proj_w, proj_b)


def kernel(x, edge_index, batch, gnn_w_rel, gnn_b_rel, gnn_w_root, proj_w, proj_b):
    s, r = _prep(x, gnn_w_rel, gnn_w_root, gnn_b_rel)
    partials = _seg_sum(s, r, edge_index)
    x3 = x.reshape(B, NPG, D)
    return _pool_project(x3, partials, proj_w, proj_b)
